# Initial kernel scaffold; baseline (speedup 1.0000x reference)
import jax
import jax.numpy as jnp
from jax.experimental import pallas as pl


def kernel(x, edge_index, lin1_w, lin1_b, cheb1_w, cheb1_b, cheb2_w, cheb2_b, lin2_w, lin2_b):
    raise NotImplementedError("write your pallas kernel here")


# speedup vs baseline: 4.8623x; 4.8623x over previous
import functools

import jax
import jax.numpy as jnp
from jax import lax
from jax.experimental import pallas as pl
from jax.experimental.pallas import tpu as pltpu
from jax.experimental.pallas import tpu_sc as plsc

_NC = 2
_NS = 16
_NW = _NC * _NS
_CH = 128


def _sc_mesh():
    return plsc.VectorSubcoreMesh(
        core_axis_name="c", subcore_axis_name="s",
        num_cores=_NC, num_subcores=_NS)


def _sc_degree(n_chunks, acc_rows, d=128):
    zr = acc_rows // _NS

    @functools.partial(
        pl.kernel,
        out_type=jax.ShapeDtypeStruct((_NC, acc_rows, d), jnp.float32),
        mesh=_sc_mesh(),
        scratch_types=[
            pltpu.VMEM((n_chunks, _CH), jnp.int32),
            pltpu.VMEM((_CH, d), jnp.float32),
            pltpu.VMEM_SHARED((acc_rows, d), jnp.float32),
        ],
    )
    def deg_kernel(src_hbm, ones_hbm, zeros_hbm, out_hbm, src_v, ones_v, acc):
        c = lax.axis_index("c")
        s = lax.axis_index("s")
        wid = c * _NS + s
        pltpu.sync_copy(src_hbm.at[wid], src_v)
        pltpu.sync_copy(ones_hbm, ones_v)
        pltpu.sync_copy(zeros_hbm.at[pl.ds(s * zr, zr)], acc.at[pl.ds(s * zr, zr)])
        plsc.subcore_barrier()

        def step(i, carry):
            pltpu.sync_copy(ones_v, acc.at[src_v.at[i]], add=True)
            return carry

        lax.fori_loop(0, n_chunks, step, 0)
        plsc.subcore_barrier()
        pltpu.sync_copy(acc.at[pl.ds(s * zr, zr)], out_hbm.at[c, pl.ds(s * zr, zr)])

    return deg_kernel


_IB = 16


def _sc_prop(d, n_chunks, acc_rows):
    zr = acc_rows // _NS
    nb = n_chunks // _IB

    @functools.partial(
        pl.kernel,
        out_type=jax.ShapeDtypeStruct((_NC, acc_rows, d), jnp.float32),
        mesh=_sc_mesh(),
        scratch_types=[
            pltpu.VMEM((_IB, _CH), jnp.int32),
            pltpu.VMEM((_IB, _CH), jnp.int32),
            pltpu.VMEM((_IB, _CH), jnp.int32),
            pltpu.VMEM((_IB, _CH), jnp.int32),
            pltpu.VMEM((_CH, d), jnp.float32),
            pltpu.VMEM((_CH, d), jnp.float32),
            pltpu.VMEM_SHARED((acc_rows, d), jnp.float32),
            pltpu.SemaphoreType.DMA,
            pltpu.SemaphoreType.DMA,
            pltpu.SemaphoreType.DMA,
        ],
    )
    def prop_kernel(src_hbm, dst_hbm, u_hbm, zeros_hbm, out_hbm,
                    srcb0, srcb1, dstb0, dstb1, rows0, rows1, acc,
                    sem0, sem1, isem):
        c = lax.axis_index("c")
        s = lax.axis_index("s")
        wid = c * _NS + s
        srcb = (srcb0, srcb1)
        dstb = (dstb0, dstb1)

        def fetch_block(b):
            lo = pl.ds(b * _IB, _IB)
            d0 = pltpu.async_copy(src_hbm.at[wid, lo], srcb[b % 2], isem)
            d1 = pltpu.async_copy(dst_hbm.at[wid, lo], dstb[b % 2], isem)
            d0.wait()
            d1.wait()

        fetch_block(0)
        pltpu.sync_copy(zeros_hbm.at[pl.ds(s * zr, zr)], acc.at[pl.ds(s * zr, zr)])
        plsc.subcore_barrier()

        pltpu.async_copy(u_hbm.at[srcb[0].at[0]], rows0, sem0)
        pltpu.async_copy(u_hbm.at[srcb[0].at[1]], rows1, sem1)

        for b in range(nb):
            cur_src = srcb[b % 2]
            cur_dst = dstb[b % 2]
            if b + 1 < nb:
                fetch_block(b + 1)
            nxt_src = srcb[(b + 1) % 2]
            last = b + 1 == nb

            def step(l, rows, sem):
                pltpu.make_async_copy(u_hbm.at[cur_src.at[l]], rows, sem).wait()
                pltpu.sync_copy(rows, acc.at[cur_dst.at[l]], add=True)
                nl = l + 2

                @pl.when(nl < _IB)
                def _():
                    pltpu.async_copy(u_hbm.at[cur_src.at[nl]], rows, sem)

                if not last:
                    @pl.when(nl >= _IB)
                    def _():
                        pltpu.async_copy(u_hbm.at[nxt_src.at[nl - _IB]],
                                         rows, sem)

            def pair(j, carry):
                step(2 * j, rows0, sem0)
                step(2 * j + 1, rows1, sem1)
                return carry

            lax.fori_loop(0, _IB // 2, pair, 0)

        plsc.subcore_barrier()
        pltpu.sync_copy(acc.at[pl.ds(s * zr, zr)], out_hbm.at[c, pl.ds(s * zr, zr)])

    return prop_kernel


def _row_spec(r, cols):
    return pl.BlockSpec((r, cols), lambda i: (i, 0))


def _full_spec(rows, cols):
    return pl.BlockSpec((rows, cols), lambda i: (0, 0))


def _core_spec(core, r, cols):
    return pl.BlockSpec((1, r, cols), lambda i, c=core: (c, i, 0))


def _tc_prologue(n, r, d_in, hidden):
    def body(x_ref, w_ref, b_ref, da_ref, db_ref, h_ref, u_ref, s_ref):
        deg = da_ref[0][:, 0:1] + db_ref[0][:, 0:1]
        pos = deg > 0.0
        sv = jnp.where(pos, lax.rsqrt(jnp.where(pos, deg, 1.0)), 0.0)
        h = jnp.dot(x_ref[...], w_ref[...], preferred_element_type=jnp.float32)
        h = jnp.maximum(h + b_ref[...], 0.0)
        sb = jnp.broadcast_to(sv, h.shape)
        h_ref[...] = h
        s_ref[...] = sb
        u_ref[...] = sb * h

    return pl.pallas_call(
        body,
        grid=(n // r,),
        in_specs=[_row_spec(r, d_in), _full_spec(d_in, hidden),
                  _full_spec(1, hidden), _core_spec(0, r, 128),
                  _core_spec(1, r, 128)],
        out_specs=[_row_spec(r, hidden)] * 3,
        out_shape=[jax.ShapeDtypeStruct((n, hidden), jnp.float32)] * 3,
    )


def _tc_mid(n, r, hidden, emb):
    def body(h_ref, ra_ref, rb_ref, s_ref, w0_ref, w1_ref, u1_ref, acc_ref):
        sv = s_ref[...]
        tx1 = -sv * (ra_ref[0] + rb_ref[0])
        u1_ref[...] = sv * tx1
        acc = jnp.dot(h_ref[...], w0_ref[...], preferred_element_type=jnp.float32)
        acc += jnp.dot(tx1, w1_ref[...], preferred_element_type=jnp.float32)
        acc_ref[...] = acc

    return pl.pallas_call(
        body,
        grid=(n // r,),
        in_specs=[_row_spec(r, hidden), _core_spec(0, r, hidden),
                  _core_spec(1, r, hidden), _row_spec(r, hidden),
                  _full_spec(hidden, emb), _full_spec(hidden, emb)],
        out_specs=[_row_spec(r, hidden), _row_spec(r, emb)],
        out_shape=[jax.ShapeDtypeStruct((n, hidden), jnp.float32),
                   jax.ShapeDtypeStruct((n, emb), jnp.float32)],
    )


def _tc_end_hidden(n, r, hidden, emb):
    def body(h_ref, ra_ref, rb_ref, s_ref, w2_ref, b_ref, acc_ref,
             hout_ref, unext_ref):
        sv = s_ref[...]
        tx2 = -2.0 * sv * (ra_ref[0] + rb_ref[0]) - h_ref[...]
        hout = acc_ref[...] + jnp.dot(tx2, w2_ref[...],
                                      preferred_element_type=jnp.float32)
        hout = jnp.maximum(hout + b_ref[...], 0.0)
        hout_ref[...] = hout
        unext_ref[...] = sv * hout

    return pl.pallas_call(
        body,
        grid=(n // r,),
        in_specs=[_row_spec(r, hidden), _core_spec(0, r, hidden),
                  _core_spec(1, r, hidden), _row_spec(r, hidden),
                  _full_spec(hidden, emb), _full_spec(1, emb),
                  _row_spec(r, emb)],
        out_specs=[_row_spec(r, emb)] * 2,
        out_shape=[jax.ShapeDtypeStruct((n, emb), jnp.float32)] * 2,
    )


def _tc_end_head(n, r, hidden, emb, n_out):
    def body(h_ref, ra_ref, rb_ref, s_ref, w2_ref, b_ref, acc_ref,
             wo_ref, bo_ref, y_ref):
        sv = s_ref[...]
        tx2 = -2.0 * sv * (ra_ref[0] + rb_ref[0]) - h_ref[...]
        hout = acc_ref[...] + jnp.dot(tx2, w2_ref[...],
                                      preferred_element_type=jnp.float32)
        hout = jnp.maximum(hout + b_ref[...], 0.0)
        logits = jnp.dot(hout, wo_ref[...], preferred_element_type=jnp.float32)
        logits += bo_ref[...]
        m = jnp.max(logits, axis=1, keepdims=True)
        e = jnp.exp(logits - m)
        y_ref[...] = e / jnp.sum(e, axis=1, keepdims=True)

    return pl.pallas_call(
        body,
        grid=(n // r,),
        in_specs=[_row_spec(r, hidden), _core_spec(0, r, hidden),
                  _core_spec(1, r, hidden), _row_spec(r, hidden),
                  _full_spec(hidden, emb), _full_spec(1, emb),
                  _row_spec(r, emb), _full_spec(emb, n_out),
                  _full_spec(1, n_out)],
        out_specs=[_row_spec(r, n_out)],
        out_shape=[jax.ShapeDtypeStruct((n, n_out), jnp.float32)],
    )


def kernel(x, edge_index, lin1_w, lin1_b, cheb1_w, cheb1_b,
           cheb2_w, cheb2_b, lin2_w, lin2_b):
    n, d_in = x.shape
    hidden = lin1_w.shape[1]
    emb1 = cheb1_w.shape[2]
    emb2 = cheb2_w.shape[2]
    n_out = lin2_w.shape[1]
    e = edge_index.shape[1]

    src = edge_index[0].astype(jnp.int32)
    dst = edge_index[1].astype(jnp.int32)

    n_chunks = -(-e // (_NW * _CH))
    n_chunks += n_chunks % 2
    ep = _NW * n_chunks * _CH
    pad = ep - e
    pad_src = jnp.zeros((pad,), jnp.int32)
    pad_dummy = jnp.full((pad,), n, jnp.int32)
    src_g = jnp.concatenate([src, pad_src]).reshape(_NW, n_chunks, _CH)
    dst_s = jnp.concatenate([dst, pad_dummy]).reshape(_NW, n_chunks, _CH)
    src_s = jnp.concatenate([src, pad_dummy]).reshape(_NW, n_chunks, _CH)

    acc_rows = -(-(n + 16) // 128) * 128
    zeros_h = jnp.zeros((acc_rows, hidden), jnp.float32)
    ones_128 = jnp.ones((_CH, 128), jnp.float32)

    degree = _sc_degree(n_chunks, acc_rows, 128)
    prop = _sc_prop(hidden, n_chunks, acc_rows)

    r = 2000
    prologue = _tc_prologue(n, r, d_in, hidden)
    mid1 = _tc_mid(n, r, hidden, emb1)
    end1 = _tc_end_hidden(n, r, hidden, emb1)
    mid2 = _tc_mid(n, r, emb1, emb2)
    end2 = _tc_end_head(n, r, emb1, emb2, n_out)

    deg2 = degree(src_s, ones_128, zeros_h)
    h, u0, s = prologue(x, lin1_w, lin1_b.reshape(1, -1), deg2, deg2)

    r1 = prop(src_g, dst_s, u0, zeros_h)
    u1, acc1 = mid1(h, r1, r1, s, cheb1_w[0], cheb1_w[1])
    r2 = prop(src_g, dst_s, u1, zeros_h)
    h2, u2 = end1(h, r2, r2, s, cheb1_w[2], cheb1_b.reshape(1, -1), acc1)

    r3 = prop(src_g, dst_s, u2, zeros_h)
    u3, acc2 = mid2(h2, r3, r3, s, cheb2_w[0], cheb2_w[1])
    r4 = prop(src_g, dst_s, u3, zeros_h)
    out = end2(h2, r4, r4, s, cheb2_w[2], cheb2_b.reshape(1, -1),
               acc2, lin2_w, lin2_b.reshape(1, -1))
    return out[0]


# speedup vs baseline: 4.8773x; 1.0031x over previous
import functools

import jax
import jax.numpy as jnp
from jax import lax
from jax.experimental import pallas as pl
from jax.experimental.pallas import tpu as pltpu
from jax.experimental.pallas import tpu_sc as plsc

_NC = 2
_NS = 16
_NW = _NC * _NS
_CH = 128


def _sc_mesh():
    return plsc.VectorSubcoreMesh(
        core_axis_name="c", subcore_axis_name="s",
        num_cores=_NC, num_subcores=_NS)


def _sc_degree(n_chunks, acc_rows, d=128):
    zr = acc_rows // _NS

    @functools.partial(
        pl.kernel,
        out_type=jax.ShapeDtypeStruct((_NC, acc_rows, d), jnp.float32),
        mesh=_sc_mesh(),
        scratch_types=[
            pltpu.VMEM((n_chunks, _CH), jnp.int32),
            pltpu.VMEM((_CH, d), jnp.float32),
            pltpu.VMEM_SHARED((acc_rows, d), jnp.float32),
        ],
    )
    def deg_kernel(src_hbm, ones_hbm, zeros_hbm, out_hbm, src_v, ones_v, acc):
        c = lax.axis_index("c")
        s = lax.axis_index("s")
        wid = c * _NS + s
        pltpu.sync_copy(src_hbm.at[wid], src_v)
        pltpu.sync_copy(ones_hbm, ones_v)
        pltpu.sync_copy(zeros_hbm.at[pl.ds(s * zr, zr)], acc.at[pl.ds(s * zr, zr)])
        plsc.subcore_barrier()

        def step(i, carry):
            pltpu.sync_copy(ones_v, acc.at[src_v.at[i]], add=True)
            return carry

        lax.fori_loop(0, n_chunks, step, 0)
        plsc.subcore_barrier()
        pltpu.sync_copy(acc.at[pl.ds(s * zr, zr)], out_hbm.at[c, pl.ds(s * zr, zr)])

    return deg_kernel


_NB = 2
_BI = 16


def _sc_prop(d, n_chunks, acc_rows):
    zr = acc_rows // _NS
    n_blk = n_chunks // _BI

    @functools.partial(
        pl.kernel,
        out_type=jax.ShapeDtypeStruct((_NC, acc_rows, d), jnp.float32),
        mesh=_sc_mesh(),
        scratch_types=[
            pltpu.VMEM((2, _BI, _CH), jnp.int32),
            pltpu.VMEM((2, _BI, _CH), jnp.int32),
        ] + [pltpu.VMEM((_CH, d), jnp.float32) for _ in range(_NB)] + [
            pltpu.VMEM_SHARED((acc_rows, d), jnp.float32),
        ] + [pltpu.SemaphoreType.DMA for _ in range(_NB + 2)],
    )
    def prop_kernel(src_hbm, dst_hbm, u_hbm, zeros_hbm, out_hbm,
                    src_b, dst_b, *rest):
        rows = rest[:_NB]
        acc = rest[_NB]
        sems = rest[_NB + 1:_NB + 1 + _NB]
        sem_is, sem_id = rest[_NB + 1 + _NB:]
        c = lax.axis_index("c")
        s = lax.axis_index("s")
        wid = c * _NS + s

        pltpu.sync_copy(src_hbm.at[wid, pl.ds(0, _BI)], src_b.at[0])
        pltpu.sync_copy(dst_hbm.at[wid, pl.ds(0, _BI)], dst_b.at[0])
        pltpu.sync_copy(zeros_hbm.at[pl.ds(s * zr, zr)], acc.at[pl.ds(s * zr, zr)])
        plsc.subcore_barrier()

        for k in range(_NB):
            pltpu.async_copy(u_hbm.at[src_b.at[0, k]], rows[k], sems[k])

        def blk(b, carry):
            cur = lax.rem(b, 2)
            nxt = lax.rem(b + 1, 2)
            nb = b + 1

            @pl.when(nb < n_blk)
            def _():
                pltpu.async_copy(src_hbm.at[wid, pl.ds(nb * _BI, _BI)],
                                 src_b.at[nxt], sem_is)
                pltpu.async_copy(dst_hbm.at[wid, pl.ds(nb * _BI, _BI)],
                                 dst_b.at[nxt], sem_id)

            for j in range(_BI):
                k = j % _NB
                pltpu.make_async_copy(u_hbm.at[src_b.at[cur, j]], rows[k],
                                      sems[k]).wait()
                pltpu.sync_copy(rows[k], acc.at[dst_b.at[cur, j]], add=True)
                if j == _BI - _NB:
                    @pl.when(nb < n_blk)
                    def _():
                        pltpu.make_async_copy(
                            src_hbm.at[wid, pl.ds(nb * _BI, _BI)],
                            src_b.at[nxt], sem_is).wait()
                        pltpu.make_async_copy(
                            dst_hbm.at[wid, pl.ds(nb * _BI, _BI)],
                            dst_b.at[nxt], sem_id).wait()
                if j + _NB < _BI:
                    pltpu.async_copy(u_hbm.at[src_b.at[cur, j + _NB]],
                                     rows[k], sems[k])
                else:
                    @pl.when(nb < n_blk)
                    def _():
                        pltpu.async_copy(
                            u_hbm.at[src_b.at[nxt, j + _NB - _BI]],
                            rows[k], sems[k])
            return carry

        lax.fori_loop(0, n_blk, blk, 0)

        plsc.subcore_barrier()
        pltpu.sync_copy(acc.at[pl.ds(s * zr, zr)], out_hbm.at[c, pl.ds(s * zr, zr)])

    return prop_kernel


def _row_spec(r, cols):
    return pl.BlockSpec((r, cols), lambda i: (i, 0))


def _full_spec(rows, cols):
    return pl.BlockSpec((rows, cols), lambda i: (0, 0))


def _core_spec(core, r, cols):
    return pl.BlockSpec((1, r, cols), lambda i, c=core: (c, i, 0))


def _tc_prologue(n, r, d_in, hidden):
    def body(x_ref, w_ref, b_ref, da_ref, db_ref, h_ref, u_ref, s_ref):
        deg = da_ref[0][:, 0:1] + db_ref[0][:, 0:1]
        pos = deg > 0.0
        sv = jnp.where(pos, lax.rsqrt(jnp.where(pos, deg, 1.0)), 0.0)
        h = jnp.dot(x_ref[...], w_ref[...], preferred_element_type=jnp.float32)
        h = jnp.maximum(h + b_ref[...], 0.0)
        sb = jnp.broadcast_to(sv, h.shape)
        h_ref[...] = h
        s_ref[...] = sb
        u_ref[...] = sb * h

    return pl.pallas_call(
        body,
        grid=(n // r,),
        in_specs=[_row_spec(r, d_in), _full_spec(d_in, hidden),
                  _full_spec(1, hidden), _core_spec(0, r, 128),
                  _core_spec(1, r, 128)],
        out_specs=[_row_spec(r, hidden)] * 3,
        out_shape=[jax.ShapeDtypeStruct((n, hidden), jnp.float32)] * 3,
    )


def _tc_mid(n, r, hidden, emb):
    def body(h_ref, ra_ref, rb_ref, s_ref, w0_ref, w1_ref, u1_ref, acc_ref):
        sv = s_ref[...]
        tx1 = -sv * (ra_ref[0] + rb_ref[0])
        u1_ref[...] = sv * tx1
        acc = jnp.dot(h_ref[...], w0_ref[...], preferred_element_type=jnp.float32)
        acc += jnp.dot(tx1, w1_ref[...], preferred_element_type=jnp.float32)
        acc_ref[...] = acc

    return pl.pallas_call(
        body,
        grid=(n // r,),
        in_specs=[_row_spec(r, hidden), _core_spec(0, r, hidden),
                  _core_spec(1, r, hidden), _row_spec(r, hidden),
                  _full_spec(hidden, emb), _full_spec(hidden, emb)],
        out_specs=[_row_spec(r, hidden), _row_spec(r, emb)],
        out_shape=[jax.ShapeDtypeStruct((n, hidden), jnp.float32),
                   jax.ShapeDtypeStruct((n, emb), jnp.float32)],
    )


def _tc_end_hidden(n, r, hidden, emb):
    def body(h_ref, ra_ref, rb_ref, s_ref, w2_ref, b_ref, acc_ref,
             hout_ref, unext_ref):
        sv = s_ref[...]
        tx2 = -2.0 * sv * (ra_ref[0] + rb_ref[0]) - h_ref[...]
        hout = acc_ref[...] + jnp.dot(tx2, w2_ref[...],
                                      preferred_element_type=jnp.float32)
        hout = jnp.maximum(hout + b_ref[...], 0.0)
        hout_ref[...] = hout
        unext_ref[...] = sv * hout

    return pl.pallas_call(
        body,
        grid=(n // r,),
        in_specs=[_row_spec(r, hidden), _core_spec(0, r, hidden),
                  _core_spec(1, r, hidden), _row_spec(r, hidden),
                  _full_spec(hidden, emb), _full_spec(1, emb),
                  _row_spec(r, emb)],
        out_specs=[_row_spec(r, emb)] * 2,
        out_shape=[jax.ShapeDtypeStruct((n, emb), jnp.float32)] * 2,
    )


def _tc_end_head(n, r, hidden, emb, n_out):
    def body(h_ref, ra_ref, rb_ref, s_ref, w2_ref, b_ref, acc_ref,
             wo_ref, bo_ref, y_ref):
        sv = s_ref[...]
        tx2 = -2.0 * sv * (ra_ref[0] + rb_ref[0]) - h_ref[...]
        hout = acc_ref[...] + jnp.dot(tx2, w2_ref[...],
                                      preferred_element_type=jnp.float32)
        hout = jnp.maximum(hout + b_ref[...], 0.0)
        logits = jnp.dot(hout, wo_ref[...], preferred_element_type=jnp.float32)
        logits += bo_ref[...]
        m = jnp.max(logits, axis=1, keepdims=True)
        e = jnp.exp(logits - m)
        y_ref[...] = e / jnp.sum(e, axis=1, keepdims=True)

    return pl.pallas_call(
        body,
        grid=(n // r,),
        in_specs=[_row_spec(r, hidden), _core_spec(0, r, hidden),
                  _core_spec(1, r, hidden), _row_spec(r, hidden),
                  _full_spec(hidden, emb), _full_spec(1, emb),
                  _row_spec(r, emb), _full_spec(emb, n_out),
                  _full_spec(1, n_out)],
        out_specs=[_row_spec(r, n_out)],
        out_shape=[jax.ShapeDtypeStruct((n, n_out), jnp.float32)],
    )


def kernel(x, edge_index, lin1_w, lin1_b, cheb1_w, cheb1_b,
           cheb2_w, cheb2_b, lin2_w, lin2_b):
    n, d_in = x.shape
    hidden = lin1_w.shape[1]
    emb1 = cheb1_w.shape[2]
    emb2 = cheb2_w.shape[2]
    n_out = lin2_w.shape[1]
    e = edge_index.shape[1]

    src = edge_index[0].astype(jnp.int32)
    dst = edge_index[1].astype(jnp.int32)

    n_chunks = -(-e // (_NW * _CH))
    n_chunks += (-n_chunks) % _BI
    ep = _NW * n_chunks * _CH
    pad = ep - e
    pad_src = jnp.zeros((pad,), jnp.int32)
    pad_dummy = jnp.full((pad,), n, jnp.int32)
    src_g = jnp.concatenate([src, pad_src]).reshape(_NW, n_chunks, _CH)
    dst_s = jnp.concatenate([dst, pad_dummy]).reshape(_NW, n_chunks, _CH)
    src_s = jnp.concatenate([src, pad_dummy]).reshape(_NW, n_chunks, _CH)

    acc_rows = -(-(n + 16) // 128) * 128
    zeros_h = jnp.zeros((acc_rows, hidden), jnp.float32)
    ones_128 = jnp.ones((_CH, 128), jnp.float32)

    degree = _sc_degree(n_chunks, acc_rows, 128)
    prop = _sc_prop(hidden, n_chunks, acc_rows)

    r = 2000
    prologue = _tc_prologue(n, r, d_in, hidden)
    mid1 = _tc_mid(n, r, hidden, emb1)
    end1 = _tc_end_hidden(n, r, hidden, emb1)
    mid2 = _tc_mid(n, r, emb1, emb2)
    end2 = _tc_end_head(n, r, emb1, emb2, n_out)

    deg2 = degree(src_s, ones_128, zeros_h)
    h, u0, s = prologue(x, lin1_w, lin1_b.reshape(1, -1), deg2, deg2)

    r1 = prop(src_g, dst_s, u0, zeros_h)
    u1, acc1 = mid1(h, r1, r1, s, cheb1_w[0], cheb1_w[1])
    r2 = prop(src_g, dst_s, u1, zeros_h)
    h2, u2 = end1(h, r2, r2, s, cheb1_w[2], cheb1_b.reshape(1, -1), acc1)

    r3 = prop(src_g, dst_s, u2, zeros_h)
    u3, acc2 = mid2(h2, r3, r3, s, cheb2_w[0], cheb2_w[1])
    r4 = prop(src_g, dst_s, u3, zeros_h)
    out = end2(h2, r4, r4, s, cheb2_w[2], cheb2_b.reshape(1, -1),
               acc2, lin2_w, lin2_b.reshape(1, -1))
    return out[0]
